# Initial kernel scaffold; baseline (speedup 1.0000x reference)
import jax
import jax.numpy as jnp
from jax.experimental import pallas as pl


def kernel(token_ids, table):
    raise NotImplementedError("write your pallas kernel here")


# speedup vs baseline: 5.1797x; 5.1797x over previous
import functools

import jax
import jax.numpy as jnp
from jax import lax
from jax.experimental import pallas as pl
from jax.experimental.pallas import tpu as pltpu
from jax.experimental.pallas import tpu_sc as plsc

HIDDEN = 128
NC, NS = 2, 16
NW = NC * NS
CHUNK = 128


def _make_lookup(B):
    b_per_w = B // NW
    n_chunks = b_per_w // CHUNK
    mesh = plsc.VectorSubcoreMesh(
        core_axis_name="c", subcore_axis_name="s", num_cores=NC, num_subcores=NS
    )

    @functools.partial(
        pl.kernel,
        out_type=jax.ShapeDtypeStruct((B, HIDDEN), jnp.float32),
        mesh=mesh,
        scratch_types=[
            pltpu.VMEM((CHUNK,), jnp.int32),
            pltpu.VMEM((CHUNK, HIDDEN), jnp.float32),
            pltpu.SemaphoreType.DMA,
        ],
    )
    def lookup(idx_hbm, table_hbm, out_hbm, idx_v, rows_v, sem):
        wid = lax.axis_index("s") * NC + lax.axis_index("c")
        base = wid * b_per_w

        def body(i, carry):
            off = base + i * CHUNK
            pltpu.sync_copy(idx_hbm.at[pl.ds(off, CHUNK)], idx_v)
            pltpu.async_copy(table_hbm.at[idx_v], rows_v, sem).wait()
            pltpu.sync_copy(rows_v, out_hbm.at[pl.ds(off, CHUNK)])
            return carry

        lax.fori_loop(0, n_chunks, body, 0)

    return lookup


def kernel(token_ids, table):
    B_, S_ = token_ids.shape
    flat = jnp.reshape(token_ids, (-1,)).astype(jnp.int32)
    out = _make_lookup(flat.shape[0])(flat, table)
    return jnp.reshape(out, (B_, S_, HIDDEN))


# speedup vs baseline: 9.2576x; 1.7873x over previous
import functools

import jax
import jax.numpy as jnp
from jax import lax
from jax.experimental import pallas as pl
from jax.experimental.pallas import tpu as pltpu
from jax.experimental.pallas import tpu_sc as plsc

HIDDEN = 128
NC, NS = 2, 16
NW = NC * NS
CHUNK = 128
NBUF = 4
LOOK = 2


def _make_lookup(B):
    b_per_w = B // NW
    n_chunks = b_per_w // CHUNK
    mesh = plsc.VectorSubcoreMesh(
        core_axis_name="c", subcore_axis_name="s", num_cores=NC, num_subcores=NS
    )

    @functools.partial(
        pl.kernel,
        out_type=jax.ShapeDtypeStruct((B, HIDDEN), jnp.float32),
        mesh=mesh,
        scratch_types=[
            pltpu.VMEM((n_chunks, CHUNK), jnp.int32),
            pltpu.VMEM((NBUF, CHUNK, HIDDEN), jnp.float32),
            pltpu.SemaphoreType.DMA((NBUF,)),
            pltpu.SemaphoreType.DMA((NBUF,)),
        ],
    )
    def lookup(idx_hbm, table_hbm, out_hbm, idx_v, rows_v, gsem, ssem):
        wid = lax.axis_index("s") * NC + lax.axis_index("c")
        base = wid * n_chunks
        pltpu.sync_copy(idx_hbm.at[pl.ds(base, n_chunks)], idx_v)

        for j in range(LOOK):
            pltpu.async_copy(table_hbm.at[idx_v.at[j]], rows_v.at[j], gsem.at[j])

        def body(i, carry):
            slot = lax.rem(i, NBUF)
            j = i + LOOK

            @pl.when(j < n_chunks)
            def _():
                jslot = lax.rem(j, NBUF)

                @pl.when(i >= NBUF - LOOK)
                def _():
                    pltpu.make_async_copy(
                        rows_v.at[jslot], out_hbm.at[pl.ds(0, CHUNK)], ssem.at[jslot]
                    ).wait()

                pltpu.async_copy(
                    table_hbm.at[idx_v.at[j]], rows_v.at[jslot], gsem.at[jslot]
                )

            pltpu.make_async_copy(
                table_hbm.at[idx_v.at[slot]], rows_v.at[slot], gsem.at[slot]
            ).wait()
            pltpu.async_copy(
                rows_v.at[slot],
                out_hbm.at[pl.ds((base + i) * CHUNK, CHUNK)],
                ssem.at[slot],
            )
            return carry

        lax.fori_loop(0, n_chunks, body, 0)

        for b in range(NBUF):
            pltpu.make_async_copy(
                rows_v.at[b], out_hbm.at[pl.ds(0, CHUNK)], ssem.at[b]
            ).wait()

    return lookup


def kernel(token_ids, table):
    B_, S_ = token_ids.shape
    flat = jnp.reshape(token_ids, (-1, CHUNK)).astype(jnp.int32)
    out = _make_lookup(B_ * S_)(flat, table)
    return jnp.reshape(out, (B_, S_, HIDDEN))


# speedup vs baseline: 9.2949x; 1.0040x over previous
import functools

import jax
import jax.numpy as jnp
from jax import lax
from jax.experimental import pallas as pl
from jax.experimental.pallas import tpu as pltpu
from jax.experimental.pallas import tpu_sc as plsc

HIDDEN = 128
NC, NS = 2, 16
NW = NC * NS
CHUNK = 128
NBUF = 6
LOOK = 3


def _make_lookup(B):
    b_per_w = B // NW
    n_chunks = b_per_w // CHUNK
    mesh = plsc.VectorSubcoreMesh(
        core_axis_name="c", subcore_axis_name="s", num_cores=NC, num_subcores=NS
    )

    @functools.partial(
        pl.kernel,
        out_type=jax.ShapeDtypeStruct((B, HIDDEN), jnp.float32),
        mesh=mesh,
        scratch_types=[
            pltpu.VMEM((n_chunks, CHUNK), jnp.int32),
            pltpu.VMEM((NBUF, CHUNK, HIDDEN), jnp.float32),
            pltpu.SemaphoreType.DMA((NBUF,)),
            pltpu.SemaphoreType.DMA((NBUF,)),
        ],
    )
    def lookup(idx_hbm, table_hbm, out_hbm, idx_v, rows_v, gsem, ssem):
        wid = lax.axis_index("s") * NC + lax.axis_index("c")
        base = wid * n_chunks
        pltpu.sync_copy(idx_hbm.at[pl.ds(base, n_chunks)], idx_v)

        for j in range(LOOK):
            pltpu.async_copy(table_hbm.at[idx_v.at[j]], rows_v.at[j], gsem.at[j])

        def body(i, carry):
            slot = lax.rem(i, NBUF)
            j = i + LOOK

            @pl.when(j < n_chunks)
            def _():
                jslot = lax.rem(j, NBUF)

                @pl.when(i >= NBUF - LOOK)
                def _():
                    pltpu.make_async_copy(
                        rows_v.at[jslot], out_hbm.at[pl.ds(0, CHUNK)], ssem.at[jslot]
                    ).wait()

                pltpu.async_copy(
                    table_hbm.at[idx_v.at[j]], rows_v.at[jslot], gsem.at[jslot]
                )

            pltpu.make_async_copy(
                table_hbm.at[idx_v.at[slot]], rows_v.at[slot], gsem.at[slot]
            ).wait()
            pltpu.async_copy(
                rows_v.at[slot],
                out_hbm.at[pl.ds((base + i) * CHUNK, CHUNK)],
                ssem.at[slot],
            )
            return carry

        lax.fori_loop(0, n_chunks, body, 0)

        for b in range(NBUF):
            pltpu.make_async_copy(
                rows_v.at[b], out_hbm.at[pl.ds(0, CHUNK)], ssem.at[b]
            ).wait()

    return lookup


def kernel(token_ids, table):
    B_, S_ = token_ids.shape
    flat = jnp.reshape(token_ids, (-1, CHUNK)).astype(jnp.int32)
    out = _make_lookup(B_ * S_)(flat, table)
    return jnp.reshape(out, (B_, S_, HIDDEN))
